# Initial kernel scaffold; baseline (speedup 1.0000x reference)
import jax
import jax.numpy as jnp
from jax.experimental import pallas as pl


def kernel(x, input_len, diags, bias, epsilon, end_states):
    raise NotImplementedError("write your pallas kernel here")


# speedup vs baseline: 6.3006x; 6.3006x over previous
import numpy as np
import jax
import jax.numpy as jnp
from jax.experimental import pallas as pl
from jax.experimental.pallas import tpu as pltpu

ZERO = -100.0


def _sopa_kernel(x_ref, il_ref, w_ref, b_ref, eps_ref, es_ref, out_ref):
    L, Bb, D = x_ref.shape
    NP = w_ref.shape[1] // 2
    N = es_ref.shape[1]
    S = NP - N

    lane = jax.lax.broadcasted_iota(jnp.int32, (Bb, NP), 1)
    h0 = jnp.where(lane < N, 0.0, ZERO)
    sc0 = jnp.full((Bb, N), ZERO, dtype=jnp.float32)

    def body(i, carry):
        h, sc = carry
        x_t = x_ref[i]
        ts = jnp.dot(x_t, w_ref[:, :], preferred_element_type=jnp.float32)
        ts = ts + b_ref[:, :]
        tr0 = ts[:, :NP]
        tr1 = ts[:, NP:]
        shifted = jnp.concatenate(
            [jnp.full((Bb, N), ZERO, dtype=jnp.float32),
             h[:, :S] + eps_ref[:, :]], axis=1)
        after = jnp.maximum(h, shifted)
        main = jnp.concatenate(
            [jnp.zeros((Bb, N), dtype=jnp.float32),
             after[:, :S] + tr1[:, :S]], axis=1)
        h = jnp.maximum(main, after + tr0)
        ev = h[:, 0:N]
        for p in range(1, NP // N):
            ev = jnp.where(es_ref[:, :] == p, h[:, p * N:(p + 1) * N], ev)
        act = il_ref[:, :] >= i
        sc = jnp.where(act, jnp.maximum(sc, ev), sc)
        out_ref[i] = jnp.tanh(sc)
        return h, sc

    jax.lax.fori_loop(0, L, body, (h0, sc0))


def kernel(x, input_len, diags, bias, epsilon, end_states):
    L, B, D = x.shape
    N, Pm1 = epsilon.shape
    P = Pm1 + 1
    NP = N * P

    n_i = np.arange(N)
    perm = np.empty(2 * NP, dtype=np.int32)
    for s in range(2):
        for p in range(P):
            perm[s * NP + p * N + n_i] = n_i * 2 * P + s * P + p
    w = jnp.transpose(diags[perm, :], (1, 0))
    b = bias[perm, 0][None, :]
    eps_row = jnp.transpose(epsilon, (1, 0)).reshape(1, Pm1 * N)
    es_row = end_states[:, 0][None, :].astype(jnp.int32)
    il = input_len.astype(jnp.int32)[:, None]

    Bb = 128
    grid = (B // Bb,)
    out = pl.pallas_call(
        _sopa_kernel,
        grid=grid,
        in_specs=[
            pl.BlockSpec((L, Bb, D), lambda i: (0, i, 0)),
            pl.BlockSpec((Bb, 1), lambda i: (i, 0)),
            pl.BlockSpec((D, 2 * NP), lambda i: (0, 0)),
            pl.BlockSpec((1, 2 * NP), lambda i: (0, 0)),
            pl.BlockSpec((1, Pm1 * N), lambda i: (0, 0)),
            pl.BlockSpec((1, N), lambda i: (0, 0)),
        ],
        out_specs=pl.BlockSpec((L, Bb, N), lambda i: (0, i, 0)),
        out_shape=jax.ShapeDtypeStruct((L, B, N), jnp.float32),
        compiler_params=pltpu.CompilerParams(
            dimension_semantics=("arbitrary",),
        ),
    )(x, il, w, b, eps_row, es_row)
    return out


# speedup vs baseline: 6.9851x; 1.1086x over previous
import numpy as np
import jax
import jax.numpy as jnp
from jax.experimental import pallas as pl
from jax.experimental.pallas import tpu as pltpu

ZERO = -100.0


def _sopa_kernel(x_ref, il_ref, w_ref, b_ref, eps_ref, es_ref, out_ref,
                 ts_ref):
    L, Bb, D = x_ref.shape
    NP = w_ref.shape[1] // 2
    N = es_ref.shape[1]
    S = NP - N

    xf = x_ref[:, :, :].reshape(L * Bb, D)
    ts_ref[:, :] = (
        jnp.dot(xf, w_ref[:, :], preferred_element_type=jnp.float32)
        + b_ref[:, :])

    lane = jax.lax.broadcasted_iota(jnp.int32, (Bb, NP), 1)
    h0 = jnp.where(lane < N, 0.0, ZERO)
    sc0 = jnp.full((Bb, N), ZERO, dtype=jnp.float32)

    def body(i, carry):
        h, sc = carry
        ts = ts_ref[pl.ds(i * Bb, Bb), :]
        tr0 = ts[:, :NP]
        tr1 = ts[:, NP:]
        shifted = jnp.concatenate(
            [jnp.full((Bb, N), ZERO, dtype=jnp.float32),
             h[:, :S] + eps_ref[:, :]], axis=1)
        after = jnp.maximum(h, shifted)
        main = jnp.concatenate(
            [jnp.zeros((Bb, N), dtype=jnp.float32),
             after[:, :S] + tr1[:, :S]], axis=1)
        h = jnp.maximum(main, after + tr0)
        ev = h[:, 0:N]
        for p in range(1, NP // N):
            ev = jnp.where(es_ref[:, :] == p, h[:, p * N:(p + 1) * N], ev)
        act = il_ref[:, :] >= i
        sc = jnp.where(act, jnp.maximum(sc, ev), sc)
        out_ref[i] = sc
        return h, sc

    jax.lax.fori_loop(0, L, body, (h0, sc0))
    out_ref[:, :, :] = jnp.tanh(out_ref[:, :, :])


def kernel(x, input_len, diags, bias, epsilon, end_states):
    L, B, D = x.shape
    N, Pm1 = epsilon.shape
    P = Pm1 + 1
    NP = N * P

    n_i = np.arange(N)
    perm = np.empty(2 * NP, dtype=np.int32)
    for s in range(2):
        for p in range(P):
            perm[s * NP + p * N + n_i] = n_i * 2 * P + s * P + p
    w = jnp.transpose(diags[perm, :], (1, 0))
    b = bias[perm, 0][None, :]
    eps_row = jnp.transpose(epsilon, (1, 0)).reshape(1, Pm1 * N)
    es_row = end_states[:, 0][None, :].astype(jnp.int32)
    il = input_len.astype(jnp.int32)[:, None]

    Bb = 128
    grid = (B // Bb,)
    out = pl.pallas_call(
        _sopa_kernel,
        grid=grid,
        in_specs=[
            pl.BlockSpec((L, Bb, D), lambda i: (0, i, 0)),
            pl.BlockSpec((Bb, 1), lambda i: (i, 0)),
            pl.BlockSpec((D, 2 * NP), lambda i: (0, 0)),
            pl.BlockSpec((1, 2 * NP), lambda i: (0, 0)),
            pl.BlockSpec((1, Pm1 * N), lambda i: (0, 0)),
            pl.BlockSpec((1, N), lambda i: (0, 0)),
        ],
        out_specs=pl.BlockSpec((L, Bb, N), lambda i: (0, i, 0)),
        out_shape=jax.ShapeDtypeStruct((L, B, N), jnp.float32),
        scratch_shapes=[pltpu.VMEM((L * Bb, 2 * NP), jnp.float32)],
        compiler_params=pltpu.CompilerParams(
            dimension_semantics=("arbitrary",),
        ),
    )(x, il, w, b, eps_row, es_row)
    return out


# speedup vs baseline: 8.6174x; 1.2337x over previous
import numpy as np
import jax
import jax.numpy as jnp
from jax.experimental import pallas as pl
from jax.experimental.pallas import tpu as pltpu

ZERO = -100.0


def _sopa_kernel(x_ref, il_ref, w_ref, b_ref, eps_ref, es_ref, out_ref,
                 ts_ref):
    L, Bb, D = x_ref.shape
    NP = w_ref.shape[1] // 2
    N = es_ref.shape[1]
    S = NP - N

    xf = x_ref[:, :, :].reshape(L * Bb, D)
    ts_ref[:, :] = (
        jnp.dot(xf, w_ref[:, :], preferred_element_type=jnp.float32)
        + b_ref[:, :])

    lane = jax.lax.broadcasted_iota(jnp.int32, (Bb, NP), 1)
    h0 = jnp.where(lane < N, 0.0, ZERO)
    sc0 = jnp.full((Bb, N), ZERO, dtype=jnp.float32)

    def body(i, carry):
        h, sc = carry
        ts = ts_ref[pl.ds(i * Bb, Bb), :]
        tr0 = ts[:, :NP]
        tr1 = ts[:, NP:]
        shifted = jnp.concatenate(
            [jnp.full((Bb, N), ZERO, dtype=jnp.float32),
             h[:, :S] + eps_ref[:, :]], axis=1)
        after = jnp.maximum(h, shifted)
        main = jnp.concatenate(
            [jnp.zeros((Bb, N), dtype=jnp.float32),
             after[:, :S] + tr1[:, :S]], axis=1)
        h = jnp.maximum(main, after + tr0)
        ev = h[:, 0:N]
        for p in range(1, NP // N):
            ev = jnp.where(es_ref[:, :] == p, h[:, p * N:(p + 1) * N], ev)
        act = il_ref[:, :] >= i
        sc = jnp.where(act, jnp.maximum(sc, ev), sc)
        out_ref[i] = sc
        return h, sc

    jax.lax.fori_loop(0, L, body, (h0, sc0))
    out_ref[:, :, :] = jnp.tanh(out_ref[:, :, :])


def kernel(x, input_len, diags, bias, epsilon, end_states):
    L, B, D = x.shape
    N, Pm1 = epsilon.shape
    P = Pm1 + 1
    NP = N * P

    n_i = np.arange(N)
    perm = np.empty(2 * NP, dtype=np.int32)
    for s in range(2):
        for p in range(P):
            perm[s * NP + p * N + n_i] = n_i * 2 * P + s * P + p
    w = jnp.transpose(diags[perm, :], (1, 0))
    b = bias[perm, 0][None, :]
    eps_row = jnp.transpose(epsilon, (1, 0)).reshape(1, Pm1 * N)
    es_row = end_states[:, 0][None, :].astype(jnp.int32)
    il = input_len.astype(jnp.int32)[:, None]

    Bb = 256
    grid = (B // Bb,)
    out = pl.pallas_call(
        _sopa_kernel,
        grid=grid,
        in_specs=[
            pl.BlockSpec((L, Bb, D), lambda i: (0, i, 0)),
            pl.BlockSpec((Bb, 1), lambda i: (i, 0)),
            pl.BlockSpec((D, 2 * NP), lambda i: (0, 0)),
            pl.BlockSpec((1, 2 * NP), lambda i: (0, 0)),
            pl.BlockSpec((1, Pm1 * N), lambda i: (0, 0)),
            pl.BlockSpec((1, N), lambda i: (0, 0)),
        ],
        out_specs=pl.BlockSpec((L, Bb, N), lambda i: (0, i, 0)),
        out_shape=jax.ShapeDtypeStruct((L, B, N), jnp.float32),
        scratch_shapes=[pltpu.VMEM((L * Bb, 2 * NP), jnp.float32)],
        compiler_params=pltpu.CompilerParams(
            dimension_semantics=("arbitrary",),
        ),
    )(x, il, w, b, eps_row, es_row)
    return out


# speedup vs baseline: 9.2219x; 1.0701x over previous
import numpy as np
import jax
import jax.numpy as jnp
from jax.experimental import pallas as pl
from jax.experimental.pallas import tpu as pltpu

ZERO = -100.0


def _sopa_kernel(x_ref, il_ref, w_ref, b_ref, eps_ref, es_ref, out_ref,
                 ts_ref, h_ref, sc_ref):
    Lc, B, D = x_ref.shape
    NP = w_ref.shape[1] // 2
    N = es_ref.shape[1]
    S = NP - N
    l = pl.program_id(0)

    @pl.when(l == 0)
    def _init():
        lane = jax.lax.broadcasted_iota(jnp.int32, (B, NP), 1)
        h_ref[:, :] = jnp.where(lane < N, 0.0, ZERO)
        sc_ref[:, :] = jnp.full((B, N), ZERO, dtype=jnp.float32)

    xf = x_ref[:, :, :].reshape(Lc * B, D)
    ts_ref[:, :] = (
        jnp.dot(xf, w_ref[:, :], preferred_element_type=jnp.float32)
        + b_ref[:, :])

    def body(j, carry):
        h, sc = carry
        ts = ts_ref[pl.ds(j * B, B), :]
        tr0 = ts[:, :NP]
        tr1 = ts[:, NP:]
        shifted = jnp.concatenate(
            [jnp.full((B, N), ZERO, dtype=jnp.float32),
             h[:, :S] + eps_ref[:, :]], axis=1)
        after = jnp.maximum(h, shifted)
        main = jnp.concatenate(
            [jnp.zeros((B, N), dtype=jnp.float32),
             after[:, :S] + tr1[:, :S]], axis=1)
        h = jnp.maximum(main, after + tr0)
        ev = h[:, 0:N]
        for p in range(1, NP // N):
            ev = jnp.where(es_ref[:, :] == p, h[:, p * N:(p + 1) * N], ev)
        act = il_ref[:, :] >= (l * Lc + j)
        sc = jnp.where(act, jnp.maximum(sc, ev), sc)
        out_ref[j] = jnp.tanh(sc)
        return h, sc

    h, sc = jax.lax.fori_loop(
        0, Lc, body, (h_ref[:, :], sc_ref[:, :]))
    h_ref[:, :] = h
    sc_ref[:, :] = sc


def kernel(x, input_len, diags, bias, epsilon, end_states):
    L, B, D = x.shape
    N, Pm1 = epsilon.shape
    P = Pm1 + 1
    NP = N * P

    n_i = np.arange(N)
    perm = np.empty(2 * NP, dtype=np.int32)
    for s in range(2):
        for p in range(P):
            perm[s * NP + p * N + n_i] = n_i * 2 * P + s * P + p
    w = jnp.transpose(diags[perm, :], (1, 0))
    b = bias[perm, 0][None, :]
    eps_row = jnp.transpose(epsilon, (1, 0)).reshape(1, Pm1 * N)
    es_row = end_states[:, 0][None, :].astype(jnp.int32)
    il = input_len.astype(jnp.int32)[:, None]

    Lc = 10
    grid = (L // Lc,)
    out = pl.pallas_call(
        _sopa_kernel,
        grid=grid,
        in_specs=[
            pl.BlockSpec((Lc, B, D), lambda l: (l, 0, 0)),
            pl.BlockSpec((B, 1), lambda l: (0, 0)),
            pl.BlockSpec((D, 2 * NP), lambda l: (0, 0)),
            pl.BlockSpec((1, 2 * NP), lambda l: (0, 0)),
            pl.BlockSpec((1, Pm1 * N), lambda l: (0, 0)),
            pl.BlockSpec((1, N), lambda l: (0, 0)),
        ],
        out_specs=pl.BlockSpec((Lc, B, N), lambda l: (l, 0, 0)),
        out_shape=jax.ShapeDtypeStruct((L, B, N), jnp.float32),
        scratch_shapes=[
            pltpu.VMEM((Lc * B, 2 * NP), jnp.float32),
            pltpu.VMEM((B, NP), jnp.float32),
            pltpu.VMEM((B, N), jnp.float32),
        ],
        compiler_params=pltpu.CompilerParams(
            dimension_semantics=("arbitrary",),
        ),
    )(x, il, w, b, eps_row, es_row)
    return out
